# Initial kernel scaffold; baseline (speedup 1.0000x reference)
import functools

import jax
import jax.numpy as jnp
from jax import lax
from jax.experimental import pallas as pl
from jax.experimental.pallas import tpu as pltpu
from jax.experimental.pallas import tpu_sc as plsc

N = 10000
E = 320000
DI = 128
H = 64
NC = 2
NS = 16
NW = NC * NS
EPT = E // NW
K = 80
NCHUNK = EPT // K
NPT = N // NS

_mesh = plsc.VectorSubcoreMesh(core_axis_name="c", subcore_axis_name="s")


@functools.partial(
    pl.kernel,
    out_type=jax.ShapeDtypeStruct((NC, N), jnp.float32),
    mesh=_mesh,
    scratch_types=[
        pltpu.VMEM((K,), jnp.int32),
        pltpu.VMEM((K,), jnp.float32),
        pltpu.VMEM_SHARED((N,), jnp.float32),
    ],
)
def _deg_kernel(dst_h, z1_h, deg_o, dst_v, ones_v, deg_sh):
    c = lax.axis_index("c")
    s = lax.axis_index("s")
    wid = c * NS + s
    for i in range(K // 16):
        ones_v[pl.ds(i * 16, 16)] = jnp.ones((16,), jnp.float32)
    rbase = s * NPT
    pltpu.sync_copy(z1_h.at[pl.ds(rbase, NPT)], deg_sh.at[pl.ds(rbase, NPT)])
    plsc.subcore_barrier()
    ebase = wid * EPT

    def chunk(j, carry):
        pltpu.sync_copy(dst_h.at[pl.ds(ebase + j * K, K)], dst_v)
        pltpu.sync_copy(ones_v, deg_sh.at[dst_v], add=True)
        return carry

    lax.fori_loop(0, NCHUNK, chunk, 0)
    plsc.subcore_barrier()
    pltpu.sync_copy(deg_sh.at[pl.ds(rbase, NPT)], deg_o.at[c, pl.ds(rbase, NPT)])


def _make_edge_pass(with_s):
    out_type = [jax.ShapeDtypeStruct((NC, N, H), jnp.float32)]
    scratch = [
        pltpu.VMEM((K,), jnp.int32),
        pltpu.VMEM((K,), jnp.int32),
        pltpu.VMEM((K, H), jnp.float32),
        pltpu.SemaphoreType.DMA,
        pltpu.VMEM_SHARED((N, H), jnp.float32),
    ]
    if with_s:
        out_type.append(jax.ShapeDtypeStruct((NC, N), jnp.float32))
        scratch += [
            pltpu.VMEM((K,), jnp.float32),
            pltpu.SemaphoreType.DMA,
            pltpu.VMEM_SHARED((N,), jnp.float32),
        ]

    def body(args):
        if with_s:
            (hh_h, src_h, dst_h, dinv_h, z2_h, z1_h, agg_o, s_o,
             src_v, dst_v, rows_v, gsem, agg_sh, vals_v, ssem, s_sh) = args
        else:
            (hh_h, src_h, dst_h, z2_h, agg_o,
             src_v, dst_v, rows_v, gsem, agg_sh) = args
        c = lax.axis_index("c")
        s = lax.axis_index("s")
        wid = c * NS + s
        rbase = s * NPT
        pltpu.sync_copy(z2_h.at[pl.ds(rbase, NPT)], agg_sh.at[pl.ds(rbase, NPT)])
        if with_s:
            pltpu.sync_copy(z1_h.at[pl.ds(rbase, NPT)], s_sh.at[pl.ds(rbase, NPT)])
        plsc.subcore_barrier()
        ebase = wid * EPT

        def chunk(j, carry):
            off = ebase + j * K
            pltpu.sync_copy(src_h.at[pl.ds(off, K)], src_v)
            pltpu.sync_copy(dst_h.at[pl.ds(off, K)], dst_v)
            gd = pltpu.async_copy(hh_h.at[src_v], rows_v, gsem)
            if with_s:
                vd = pltpu.async_copy(dinv_h.at[dst_v], vals_v, ssem)
            gd.wait()
            pltpu.sync_copy(rows_v, agg_sh.at[dst_v], add=True)
            if with_s:
                vd.wait()
                pltpu.sync_copy(vals_v, s_sh.at[src_v], add=True)
            return carry

        lax.fori_loop(0, NCHUNK, chunk, 0)
        plsc.subcore_barrier()
        pltpu.sync_copy(agg_sh.at[pl.ds(rbase, NPT)],
                        agg_o.at[c, pl.ds(rbase, NPT)])
        if with_s:
            pltpu.sync_copy(s_sh.at[pl.ds(rbase, NPT)],
                            s_o.at[c, pl.ds(rbase, NPT)])

    def body_fn(*args):
        return body(args)

    return pl.kernel(
        body_fn,
        out_type=tuple(out_type) if with_s else out_type[0],
        mesh=_mesh,
        scratch_types=scratch,
    )


_edge_pass_s = _make_edge_pass(True)
_edge_pass = _make_edge_pass(False)


def _tc1_body(deg_ref, x_ref, w_ref, hh_ref, dinv_ref):
    d = deg_ref[...]
    dinv = lax.rsqrt(d[:, 0:1] + d[:, 1:2] + 1.0)
    hh_ref[...] = dinv * jnp.dot(x_ref[...], w_ref[...],
                                 preferred_element_type=jnp.float32)
    dinv_ref[...] = dinv


_BR = 500
_GRID = N // _BR


def _tc1(degT, x, W1):
    return pl.pallas_call(
        _tc1_body,
        grid=(_GRID,),
        in_specs=[
            pl.BlockSpec((_BR, NC), lambda i: (i, 0)),
            pl.BlockSpec((_BR, DI), lambda i: (i, 0)),
            pl.BlockSpec((DI, H), lambda i: (0, 0)),
        ],
        out_specs=[
            pl.BlockSpec((_BR, H), lambda i: (i, 0)),
            pl.BlockSpec((_BR, 1), lambda i: (i, 0)),
        ],
        out_shape=[
            jax.ShapeDtypeStruct((N, H), jnp.float32),
            jax.ShapeDtypeStruct((N, 1), jnp.float32),
        ],
    )(degT, x, W1)


def _tc2_body(agg_ref, hh_ref, dinv_ref, b_ref, w_ref, out_ref):
    a = agg_ref[0] + agg_ref[1]
    dinv = dinv_ref[...]
    h = jnp.maximum(dinv * (a + hh_ref[...]) + b_ref[...], 0.0)
    out_ref[...] = dinv * jnp.dot(h, w_ref[...],
                                  preferred_element_type=jnp.float32)


def _tc2(agg, hh, dinv, b, W):
    return pl.pallas_call(
        _tc2_body,
        grid=(_GRID,),
        in_specs=[
            pl.BlockSpec((NC, _BR, H), lambda i: (0, i, 0)),
            pl.BlockSpec((_BR, H), lambda i: (i, 0)),
            pl.BlockSpec((_BR, 1), lambda i: (i, 0)),
            pl.BlockSpec((1, H), lambda i: (0, 0)),
            pl.BlockSpec((H, H), lambda i: (0, 0)),
        ],
        out_specs=pl.BlockSpec((_BR, H), lambda i: (i, 0)),
        out_shape=jax.ShapeDtypeStruct((N, H), jnp.float32),
    )(agg, hh, dinv, b, W)


def _tc3_body(agg_ref, hh_ref, dinv_ref, s_ref, b_ref, u_ref):
    a = agg_ref[0] + agg_ref[1]
    dinv = dinv_ref[...]
    h2 = jnp.maximum(dinv * (a + hh_ref[...]) + b_ref[...], 0.0)
    sv = s_ref[...]
    w = dinv * (sv[:, 0:1] + sv[:, 1:2] + dinv)

    @pl.when(pl.program_id(0) == 0)
    def _():
        u_ref[...] = jnp.zeros_like(u_ref)

    u_ref[...] += jnp.sum(w * h2, axis=0, keepdims=True)


def _tc3(agg, hh, dinv, sT, b):
    return pl.pallas_call(
        _tc3_body,
        grid=(_GRID,),
        in_specs=[
            pl.BlockSpec((NC, _BR, H), lambda i: (0, i, 0)),
            pl.BlockSpec((_BR, H), lambda i: (i, 0)),
            pl.BlockSpec((_BR, 1), lambda i: (i, 0)),
            pl.BlockSpec((_BR, NC), lambda i: (i, 0)),
            pl.BlockSpec((1, H), lambda i: (0, 0)),
        ],
        out_specs=pl.BlockSpec((1, H), lambda i: (0, 0)),
        out_shape=jax.ShapeDtypeStruct((1, H), jnp.float32),
    )(agg, hh, dinv, sT, b)


def _tc4_body(u_ref, w3_ref, b3_ref, wc1_ref, bc1_ref, wc2_ref, bc2_ref, o_ref):
    g = jnp.dot(u_ref[...], w3_ref[...],
                preferred_element_type=jnp.float32) * (1.0 / N) + b3_ref[...]
    z = jnp.maximum(jnp.dot(g, wc1_ref[...],
                            preferred_element_type=jnp.float32) + bc1_ref[...], 0.0)
    o_ref[...] = jnp.dot(z, wc2_ref[...],
                         preferred_element_type=jnp.float32) + bc2_ref[...]


def _tc4(u, W3, b3, Wc1, bc1, Wc2, bc2):
    return pl.pallas_call(
        _tc4_body,
        out_shape=jax.ShapeDtypeStruct((1, 5), jnp.float32),
    )(u, W3, b3, Wc1, bc1, Wc2, bc2)


@jax.jit
def kernel(x, edge_index, W1, b1, W2, b2, W3, b3, Wc1, bc1, Wc2, bc2):
    src = edge_index[0].astype(jnp.int32)
    dst = edge_index[1].astype(jnp.int32)
    z1 = jnp.zeros((N,), jnp.float32)
    z2 = jnp.zeros((N, H), jnp.float32)

    degp = _deg_kernel(dst, z1)
    hh1p, dinv = _tc1(degp.T, x, W1)
    agg1, sp = _edge_pass_s(hh1p, src, dst, dinv[:, 0], z2, z1)
    hh2p = _tc2(agg1, hh1p, dinv, b1.reshape(1, H), W2)
    agg2 = _edge_pass(hh2p, src, dst, z2)
    u = _tc3(agg2, hh2p, dinv, sp.T, b2.reshape(1, H))
    return _tc4(u, W3, b3.reshape(1, H), Wc1,
                bc1.reshape(1, H // 2), Wc2, bc2.reshape(1, 5))


# speedup vs baseline: 17.0287x; 17.0287x over previous
import functools

import jax
import jax.numpy as jnp
from jax import lax
from jax.experimental import pallas as pl
from jax.experimental.pallas import tpu as pltpu
from jax.experimental.pallas import tpu_sc as plsc

N = 10000
E = 320000
DI = 128
H = 64
NC = 2
NS = 16
NW = NC * NS
K = 128
NCH = E // K
CH_BASE = NCH // NW
CH_EXTRA = NCH - CH_BASE * NW
SPAN = 640
NP = NS * SPAN

_mesh = plsc.VectorSubcoreMesh(core_axis_name="c", subcore_axis_name="s")


def _striped(src_fn, dst_fn, s):
    off = pl.multiple_of(s * SPAN, 128)
    pltpu.sync_copy(src_fn(off, SPAN), dst_fn(off, SPAN))


def _chunk_range(wid):
    start = wid * CH_BASE + jnp.minimum(wid, CH_EXTRA)
    cnt = CH_BASE + jnp.where(wid < CH_EXTRA, 1, 0)
    return start, cnt


@functools.partial(
    pl.kernel,
    out_type=jax.ShapeDtypeStruct((NC, NP), jnp.float32),
    mesh=_mesh,
    scratch_types=[
        pltpu.VMEM((K,), jnp.int32),
        pltpu.VMEM((K,), jnp.float32),
        pltpu.VMEM_SHARED((NP,), jnp.float32),
    ],
    compiler_params=pltpu.CompilerParams(use_tc_tiling_on_sc=False),
)
def _deg_kernel(dst_h, z1_h, deg_o, dst_v, ones_v, deg_sh):
    c = lax.axis_index("c")
    s = lax.axis_index("s")
    wid = c * NS + s
    for i in range(K // 16):
        ones_v[pl.ds(i * 16, 16)] = jnp.ones((16,), jnp.float32)
    _striped(lambda o, n: z1_h.at[pl.ds(o, n)],
             lambda o, n: deg_sh.at[pl.ds(o, n)], s)
    plsc.subcore_barrier()
    start, cnt = _chunk_range(wid)

    def chunk(j, carry):
        off = pl.multiple_of((start + j) * K, 128)
        pltpu.sync_copy(dst_h.at[pl.ds(off, K)], dst_v)
        pltpu.sync_copy(ones_v, deg_sh.at[dst_v], add=True)
        return carry

    lax.fori_loop(0, cnt, chunk, 0)
    plsc.subcore_barrier()
    _striped(lambda o, n: deg_sh.at[pl.ds(o, n)],
             lambda o, n: deg_o.at[c, pl.ds(o, n)], s)


def _make_edge_pass(with_s):
    out_type = [jax.ShapeDtypeStruct((NC, NP, H), jnp.float32)]
    scratch = [
        pltpu.VMEM((K,), jnp.int32),
        pltpu.VMEM((K,), jnp.int32),
        pltpu.VMEM((K, H), jnp.float32),
        pltpu.SemaphoreType.DMA,
        pltpu.VMEM_SHARED((NP, H), jnp.float32),
    ]
    if with_s:
        out_type.append(jax.ShapeDtypeStruct((NC, NP), jnp.float32))
        scratch += [
            pltpu.VMEM((K,), jnp.float32),
            pltpu.SemaphoreType.DMA,
            pltpu.VMEM_SHARED((NP,), jnp.float32),
        ]

    def body(args):
        if with_s:
            (hh_h, src_h, dst_h, dinv_h, z2_h, z1_h, agg_o, s_o,
             src_v, dst_v, rows_v, gsem, agg_sh, vals_v, ssem, s_sh) = args
        else:
            (hh_h, src_h, dst_h, z2_h, agg_o,
             src_v, dst_v, rows_v, gsem, agg_sh) = args
        c = lax.axis_index("c")
        s = lax.axis_index("s")
        wid = c * NS + s
        _striped(lambda o, n: z2_h.at[pl.ds(o, n)],
                 lambda o, n: agg_sh.at[pl.ds(o, n)], s)
        if with_s:
            _striped(lambda o, n: z1_h.at[pl.ds(o, n)],
                     lambda o, n: s_sh.at[pl.ds(o, n)], s)
        plsc.subcore_barrier()
        start, cnt = _chunk_range(wid)

        def chunk(j, carry):
            off = pl.multiple_of((start + j) * K, 128)
            pltpu.sync_copy(src_h.at[pl.ds(off, K)], src_v)
            pltpu.sync_copy(dst_h.at[pl.ds(off, K)], dst_v)
            gd = pltpu.async_copy(hh_h.at[src_v], rows_v, gsem)
            if with_s:
                vd = pltpu.async_copy(dinv_h.at[dst_v], vals_v, ssem)
            gd.wait()
            pltpu.sync_copy(rows_v, agg_sh.at[dst_v], add=True)
            if with_s:
                vd.wait()
                pltpu.sync_copy(vals_v, s_sh.at[src_v], add=True)
            return carry

        lax.fori_loop(0, cnt, chunk, 0)
        plsc.subcore_barrier()
        _striped(lambda o, n: agg_sh.at[pl.ds(o, n)],
                 lambda o, n: agg_o.at[c, pl.ds(o, n)], s)
        if with_s:
            _striped(lambda o, n: s_sh.at[pl.ds(o, n)],
                     lambda o, n: s_o.at[c, pl.ds(o, n)], s)

    def body_fn(*args):
        return body(args)

    return pl.kernel(
        body_fn,
        out_type=tuple(out_type) if with_s else out_type[0],
        mesh=_mesh,
        scratch_types=scratch,
        compiler_params=pltpu.CompilerParams(use_tc_tiling_on_sc=False),
    )


_edge_pass_s = _make_edge_pass(True)
_edge_pass = _make_edge_pass(False)


def _tc1_body(deg_ref, x_ref, w_ref, hh_ref, dinv_ref):
    d = deg_ref[...]
    dinv = lax.rsqrt(d[:, 0:1] + d[:, 1:2] + 1.0)
    hh_ref[...] = dinv * jnp.dot(x_ref[...], w_ref[...],
                                 preferred_element_type=jnp.float32)
    dinv_ref[...] = dinv


_BR = 1000
_GRID = N // _BR


def _tc1(degT, x, W1):
    return pl.pallas_call(
        _tc1_body,
        grid=(_GRID,),
        in_specs=[
            pl.BlockSpec((_BR, NC), lambda i: (i, 0)),
            pl.BlockSpec((_BR, DI), lambda i: (i, 0)),
            pl.BlockSpec((DI, H), lambda i: (0, 0)),
        ],
        out_specs=[
            pl.BlockSpec((_BR, H), lambda i: (i, 0)),
            pl.BlockSpec((_BR, 1), lambda i: (i, 0)),
        ],
        out_shape=[
            jax.ShapeDtypeStruct((N, H), jnp.float32),
            jax.ShapeDtypeStruct((N, 1), jnp.float32),
        ],
    )(degT, x, W1)


def _tc2_body(agg_ref, hh_ref, dinv_ref, b_ref, w_ref, out_ref):
    a = agg_ref[0] + agg_ref[1]
    dinv = dinv_ref[...]
    h = jnp.maximum(dinv * (a + hh_ref[...]) + b_ref[...], 0.0)
    out_ref[...] = dinv * jnp.dot(h, w_ref[...],
                                  preferred_element_type=jnp.float32)


def _tc2(agg, hh, dinv, b, W):
    return pl.pallas_call(
        _tc2_body,
        grid=(_GRID,),
        in_specs=[
            pl.BlockSpec((NC, _BR, H), lambda i: (0, i, 0)),
            pl.BlockSpec((_BR, H), lambda i: (i, 0)),
            pl.BlockSpec((_BR, 1), lambda i: (i, 0)),
            pl.BlockSpec((1, H), lambda i: (0, 0)),
            pl.BlockSpec((H, H), lambda i: (0, 0)),
        ],
        out_specs=pl.BlockSpec((_BR, H), lambda i: (i, 0)),
        out_shape=jax.ShapeDtypeStruct((N, H), jnp.float32),
    )(agg, hh, dinv, b, W)


def _tc3_body(agg_ref, hh_ref, dinv_ref, s_ref, b_ref, u_ref):
    a = agg_ref[0] + agg_ref[1]
    dinv = dinv_ref[...]
    h2 = jnp.maximum(dinv * (a + hh_ref[...]) + b_ref[...], 0.0)
    sv = s_ref[...]
    w = dinv * (sv[:, 0:1] + sv[:, 1:2] + dinv)

    @pl.when(pl.program_id(0) == 0)
    def _():
        u_ref[...] = jnp.zeros_like(u_ref)

    u_ref[...] += jnp.sum(w * h2, axis=0, keepdims=True)


def _tc3(agg, hh, dinv, sT, b):
    return pl.pallas_call(
        _tc3_body,
        grid=(_GRID,),
        in_specs=[
            pl.BlockSpec((NC, _BR, H), lambda i: (0, i, 0)),
            pl.BlockSpec((_BR, H), lambda i: (i, 0)),
            pl.BlockSpec((_BR, 1), lambda i: (i, 0)),
            pl.BlockSpec((_BR, NC), lambda i: (i, 0)),
            pl.BlockSpec((1, H), lambda i: (0, 0)),
        ],
        out_specs=pl.BlockSpec((1, H), lambda i: (0, 0)),
        out_shape=jax.ShapeDtypeStruct((1, H), jnp.float32),
    )(agg, hh, dinv, sT, b)


def _tc4_body(u_ref, w3_ref, b3_ref, wc1_ref, bc1_ref, wc2_ref, bc2_ref, o_ref):
    g = jnp.dot(u_ref[...], w3_ref[...],
                preferred_element_type=jnp.float32) * (1.0 / N) + b3_ref[...]
    z = jnp.maximum(jnp.dot(g, wc1_ref[...],
                            preferred_element_type=jnp.float32) + bc1_ref[...], 0.0)
    o_ref[...] = jnp.dot(z, wc2_ref[...],
                         preferred_element_type=jnp.float32) + bc2_ref[...]


def _tc4(u, W3, b3, Wc1, bc1, Wc2, bc2):
    return pl.pallas_call(
        _tc4_body,
        out_shape=jax.ShapeDtypeStruct((1, 5), jnp.float32),
    )(u, W3, b3, Wc1, bc1, Wc2, bc2)


@jax.jit
def kernel(x, edge_index, W1, b1, W2, b2, W3, b3, Wc1, bc1, Wc2, bc2):
    src = edge_index[0].astype(jnp.int32)
    dst = edge_index[1].astype(jnp.int32)
    z1 = jnp.zeros((NP,), jnp.float32)
    z2 = jnp.zeros((NP, H), jnp.float32)

    degp = _deg_kernel(dst, z1)[:, :N]
    hh1p, dinv = _tc1(degp.T, x, W1)
    agg1, sp = _edge_pass_s(hh1p, src, dst, dinv[:, 0], z2, z1)
    agg1 = agg1[:, :N]
    sp = sp[:, :N]
    hh2p = _tc2(agg1, hh1p, dinv, b1.reshape(1, H), W2)
    agg2 = _edge_pass(hh2p, src, dst, z2)[:, :N]
    u = _tc3(agg2, hh2p, dinv, sp.T, b2.reshape(1, H))
    return _tc4(u, W3, b3.reshape(1, H), Wc1,
                bc1.reshape(1, H // 2), Wc2, bc2.reshape(1, 5))
